# baseline (speedup 1.0000x reference)
import functools

import jax
import jax.numpy as jnp
from jax import lax
from jax.experimental import pallas as pl
from jax.experimental.pallas import tpu as pltpu
from jax.experimental.pallas import tpu_sc as plsc

_N_FIELDS = 26
_VOCAB = 100000
_DIM = 64
_PAD_DIM = 128
_BATCH = 4096
_BAG = 20
_NW = 32
_BAGS_PER_W = _BATCH // _NW
_CHUNK_BAGS = 16
_N_CHUNKS_F = _BAGS_PER_W // _CHUNK_BAGS
_ROWS_PER_CHUNK = _CHUNK_BAGS * _BAG
_SLICES = 4
_SLICE_LEN = _ROWS_PER_CHUNK // _SLICES
_T_TOTAL = _N_FIELDS * _N_CHUNKS_F


def _embedding_bag_sc(gidx, w_pad):
    mesh = plsc.VectorSubcoreMesh(core_axis_name="c", subcore_axis_name="s")

    @functools.partial(
        pl.kernel,
        mesh=mesh,
        out_type=jax.ShapeDtypeStruct((_N_FIELDS * _BATCH, _DIM), jnp.float32),
        scratch_types=[
            pltpu.VMEM((_SLICES, _SLICE_LEN), jnp.int32),
            pltpu.VMEM((_SLICES, _SLICE_LEN), jnp.int32),
            pltpu.VMEM((_ROWS_PER_CHUNK, _PAD_DIM), jnp.float32),
            pltpu.VMEM((_ROWS_PER_CHUNK, _PAD_DIM), jnp.float32),
            pltpu.VMEM((_CHUNK_BAGS, _DIM), jnp.float32),
            pltpu.VMEM((_CHUNK_BAGS, _DIM), jnp.float32),
            pltpu.SemaphoreType.DMA,
            pltpu.SemaphoreType.DMA,
            pltpu.SemaphoreType.DMA,
            pltpu.SemaphoreType.DMA,
            pltpu.SemaphoreType.DMA,
            pltpu.SemaphoreType.DMA,
        ],
    )
    def k(gidx_hbm, w_hbm, out_hbm,
          ib0, ib1, rb0, rb1, ob0, ob1,
          isem0, isem1, gsem0, gsem1, osem0, osem1):
        wid = lax.axis_index("s") * 2 + lax.axis_index("c")
        ibs = (ib0, ib1)
        rbs = (rb0, rb1)
        obs = (ob0, ob1)
        isems = (isem0, isem1)
        gsems = (gsem0, gsem1)
        osems = (osem0, osem1)

        def fire_idx(t, p):
            kk = t // _N_CHUNKS_F
            cc = t % _N_CHUNKS_F
            pltpu.async_copy(gidx_hbm.at[kk, wid, cc], ibs[p], isems[p])

        def drain_idx(p):
            pltpu.make_async_copy(gidx_hbm.at[0, 0, 0], ibs[p], isems[p]).wait()

        def fire_gathers(p):
            for i in range(_SLICES):
                pltpu.async_copy(
                    w_hbm.at[ibs[p].at[i]],
                    rbs[p].at[pl.ds(i * _SLICE_LEN, _SLICE_LEN)],
                    gsems[p],
                )

        def drain_gathers(p):
            pltpu.make_async_copy(
                w_hbm.at[pl.ds(0, _ROWS_PER_CHUNK)], rbs[p], gsems[p]
            ).wait()

        def fire_out(t, p):
            kk = t // _N_CHUNKS_F
            cc = t % _N_CHUNKS_F
            base = kk * _BATCH + wid * _BAGS_PER_W + cc * _CHUNK_BAGS
            pltpu.async_copy(
                obs[p], out_hbm.at[pl.ds(base, _CHUNK_BAGS)], osems[p]
            )

        def drain_out(p):
            pltpu.make_async_copy(
                obs[p], out_hbm.at[pl.ds(0, _CHUNK_BAGS)], osems[p]
            ).wait()

        def reduce_chunk(p):
            rows = rbs[p]
            outb = obs[p]

            def body(b, carry):
                base = b * _BAG
                accs = [rows[base, pl.ds(tt * 16, 16)]
                        for tt in range(_DIM // 16)]
                for j in range(1, _BAG):
                    for tt in range(_DIM // 16):
                        accs[tt] = accs[tt] + rows[base + j,
                                                   pl.ds(tt * 16, 16)]
                for tt in range(_DIM // 16):
                    outb[b, pl.ds(tt * 16, 16)] = accs[tt]
                return carry

            lax.fori_loop(0, _CHUNK_BAGS, body, 0, unroll=False)

        def step(t, p):
            drain_gathers(p)

            @pl.when(t + 2 < _T_TOTAL)
            def _():
                fire_idx(t + 2, p)

            @pl.when(t + 1 < _T_TOTAL)
            def _():
                drain_idx(1 - p)
                fire_gathers(1 - p)

            @pl.when(t >= 2)
            def _():
                drain_out(p)

            reduce_chunk(p)
            fire_out(t, p)

        fire_idx(0, 0)
        fire_idx(1, 1)
        drain_idx(0)
        fire_gathers(0)

        def loop_body(u, carry):
            step(2 * u, 0)
            step(2 * u + 1, 1)
            return carry

        lax.fori_loop(0, _T_TOTAL // 2, loop_body, 0, unroll=False)
        drain_out(0)
        drain_out(1)

    return k(gidx, w_pad)


def kernel(indices, offsets, W):
    del offsets
    gidx = indices + (jnp.arange(_N_FIELDS, dtype=jnp.int32) * _VOCAB)[:, None]
    gidx = gidx.reshape(_N_FIELDS, _NW, _N_CHUNKS_F, _SLICES, _SLICE_LEN)
    w_pad = jnp.pad(W, ((0, 0), (0, 0), (0, _PAD_DIM - _DIM))).reshape(
        _N_FIELDS * _VOCAB, _PAD_DIM)
    out = _embedding_bag_sc(gidx, w_pad)
    return out.reshape(_N_FIELDS, _BATCH, _DIM)


# speedup vs baseline: 1.0271x; 1.0271x over previous
import functools

import jax
import jax.numpy as jnp
from jax import lax
from jax.experimental import pallas as pl
from jax.experimental.pallas import tpu as pltpu
from jax.experimental.pallas import tpu_sc as plsc

_N_FIELDS = 26
_VOCAB = 100000
_DIM = 64
_BATCH = 4096
_BAG = 20
_NW = 32
_BAGS_PER_W = _BATCH // _NW
_CHUNK_BAGS = 32
_N_CHUNKS_F = _BAGS_PER_W // _CHUNK_BAGS
_ROWS_PER_CHUNK = _CHUNK_BAGS * _BAG
_SLICES = _ROWS_PER_CHUNK // 128
_T_TOTAL = _N_FIELDS * _N_CHUNKS_F
_IDX_ROWS = _BATCH * _BAG // 128


def _embedding_bag_sc(idxr, w3):
    mesh = plsc.VectorSubcoreMesh(core_axis_name="c", subcore_axis_name="s")

    @functools.partial(
        pl.kernel,
        mesh=mesh,
        out_type=jax.ShapeDtypeStruct(
            (_N_FIELDS * _BATCH // 2, 2 * _DIM), jnp.float32),
        compiler_params=pltpu.CompilerParams(use_tc_tiling_on_sc=False),
        scratch_types=[
            pltpu.VMEM((_SLICES, 128), jnp.int32),
            pltpu.VMEM((_SLICES, 128), jnp.int32),
            pltpu.VMEM((_ROWS_PER_CHUNK, _DIM), jnp.float32),
            pltpu.VMEM((_ROWS_PER_CHUNK, _DIM), jnp.float32),
            pltpu.VMEM((_CHUNK_BAGS // 2, 2 * _DIM), jnp.float32),
            pltpu.VMEM((_CHUNK_BAGS // 2, 2 * _DIM), jnp.float32),
            pltpu.SemaphoreType.DMA,
            pltpu.SemaphoreType.DMA,
            pltpu.SemaphoreType.DMA,
            pltpu.SemaphoreType.DMA,
            pltpu.SemaphoreType.DMA,
            pltpu.SemaphoreType.DMA,
        ],
    )
    def k(idx_hbm, w_hbm, out_hbm,
          ib0, ib1, rb0, rb1, ob0, ob1,
          isem0, isem1, gsem0, gsem1, osem0, osem1):
        wid = lax.axis_index("s") * 2 + lax.axis_index("c")
        ibs = (ib0, ib1)
        rbs = (rb0, rb1)
        obs = (ob0, ob1)
        isems = (isem0, isem1)
        gsems = (gsem0, gsem1)
        osems = (osem0, osem1)

        def fire_idx(t, p):
            kk = t // _N_CHUNKS_F
            cc = t % _N_CHUNKS_F
            row0 = wid * (_SLICES * _N_CHUNKS_F) + cc * _SLICES
            pltpu.async_copy(
                idx_hbm.at[kk, pl.ds(row0, _SLICES)], ibs[p], isems[p])

        def drain_idx(p):
            pltpu.make_async_copy(
                idx_hbm.at[0, pl.ds(0, _SLICES)], ibs[p], isems[p]).wait()

        def fire_gathers(t, p):
            kk = t // _N_CHUNKS_F
            for i in range(_SLICES):
                pltpu.async_copy(
                    w_hbm.at[kk].at[ibs[p].at[i]],
                    rbs[p].at[pl.ds(i * 128, 128)],
                    gsems[p],
                )

        def drain_gathers(p):
            pltpu.make_async_copy(
                w_hbm.at[0].at[pl.ds(0, _ROWS_PER_CHUNK)], rbs[p], gsems[p]
            ).wait()

        def fire_out(t, p):
            kk = t // _N_CHUNKS_F
            cc = t % _N_CHUNKS_F
            base = (kk * _BATCH + wid * _BAGS_PER_W + cc * _CHUNK_BAGS) // 2
            pltpu.async_copy(
                obs[p], out_hbm.at[pl.ds(base, _CHUNK_BAGS // 2)], osems[p]
            )

        def drain_out(p):
            pltpu.make_async_copy(
                obs[p], out_hbm.at[pl.ds(0, _CHUNK_BAGS // 2)], osems[p]
            ).wait()

        def reduce_chunk(p):
            rows = rbs[p]
            outb = obs[p]

            def body(b, carry):
                base = b * _BAG
                half = (b % 2) * _DIM
                accs = [rows[base, pl.ds(tt * 16, 16)]
                        for tt in range(_DIM // 16)]
                for j in range(1, _BAG):
                    for tt in range(_DIM // 16):
                        accs[tt] = accs[tt] + rows[base + j,
                                                   pl.ds(tt * 16, 16)]
                for tt in range(_DIM // 16):
                    outb[b // 2, pl.ds(half + tt * 16, 16)] = accs[tt]
                return carry

            lax.fori_loop(0, _CHUNK_BAGS, body, 0, unroll=False)

        def step(t, p):
            drain_gathers(p)

            @pl.when(t + 2 < _T_TOTAL)
            def _():
                fire_idx(t + 2, p)

            @pl.when(t + 1 < _T_TOTAL)
            def _():
                drain_idx(1 - p)
                fire_gathers(t + 1, 1 - p)

            @pl.when(t >= 2)
            def _():
                drain_out(p)

            reduce_chunk(p)
            fire_out(t, p)

        fire_idx(0, 0)
        fire_idx(1, 1)
        drain_idx(0)
        fire_gathers(0, 0)

        def loop_body(u, carry):
            step(2 * u, 0)
            step(2 * u + 1, 1)
            return carry

        lax.fori_loop(0, _T_TOTAL // 2, loop_body, 0, unroll=False)
        drain_out(0)
        drain_out(1)

    return k(idxr, w3)


def kernel(indices, offsets, W):
    del offsets
    idxr = indices.reshape(_N_FIELDS, _IDX_ROWS, 128)
    out = _embedding_bag_sc(idxr, W)
    return out.reshape(_N_FIELDS, _BATCH, _DIM)
